# baseline (speedup 1.0000x reference)
import functools
import jax
import jax.numpy as jnp
from jax import lax
from jax.experimental import pallas as pl
from jax.experimental.pallas import tpu as pltpu
from jax.experimental.pallas import tpu_sc as plsc

NC = 2
NS = 16
NW = NC * NS
LN = 16
G = 128


def _round_up(a, m):
    return ((a + m - 1) // m) * m


def _sc_mesh():
    return plsc.VectorSubcoreMesh(core_axis_name="c", subcore_axis_name="s")


_SC_PARAMS = pltpu.CompilerParams(needs_layout_passes=False)


def _zero_vec(ref, n):
    z = jnp.zeros((LN,), jnp.float32)

    def body(i, _):
        ref[pl.ds(i * LN, LN)] = z
        return 0

    lax.fori_loop(0, n // LN, body, 0)


def _deg_body(NP, EP, CP, dst_hbm, bat_hbm, degacc, cntacc, dst_v, bat_v, deg_v, cnt_v):
    c = lax.axis_index("c")
    s = lax.axis_index("s")
    wid = s * NC + c
    _zero_vec(deg_v, NP)
    _zero_vec(cnt_v, CP)
    ones = jnp.ones((LN,), jnp.float32)

    epw = EP // NW
    pltpu.sync_copy(dst_hbm.at[pl.ds(wid * epw, epw)], dst_v)

    def eb(i, _):
        d16 = dst_v[pl.ds(i * LN, LN)]
        plsc.addupdate_scatter(deg_v, [d16], ones)
        return 0

    lax.fori_loop(0, epw // LN, eb, 0)

    npw = NP // NW
    pltpu.sync_copy(bat_hbm.at[pl.ds(wid * npw, npw)], bat_v)

    def bb(i, _):
        b16 = bat_v[pl.ds(i * LN, LN)]
        plsc.addupdate_scatter(cnt_v, [b16], ones)
        return 0

    lax.fori_loop(0, npw // LN, bb, 0)

    pltpu.sync_copy(deg_v, degacc.at[wid])
    pltpu.sync_copy(cnt_v, cntacc.at[wid])


def _narrow_body(NP, EP, F, src_hbm, dst_hbm, tabs_hbm, pacc, *scratch):
    src_v, dst_v = scratch[0], scratch[1]
    tab_vs = scratch[2:2 + F]
    acc_vs = scratch[2 + F:2 + 2 * F]
    c = lax.axis_index("c")
    s = lax.axis_index("s")
    wid = s * NC + c
    for f in range(F):
        pltpu.sync_copy(tabs_hbm.at[f], tab_vs[f])
        _zero_vec(acc_vs[f], NP)

    epw = EP // NW
    pltpu.sync_copy(src_hbm.at[pl.ds(wid * epw, epw)], src_v)
    pltpu.sync_copy(dst_hbm.at[pl.ds(wid * epw, epw)], dst_v)

    def eb(i, _):
        s16 = src_v[pl.ds(i * LN, LN)]
        d16 = dst_v[pl.ds(i * LN, LN)]
        for f in range(F):
            g = plsc.load_gather(tab_vs[f], [s16])
            plsc.addupdate_scatter(acc_vs[f], [d16], g)
        return 0

    lax.fori_loop(0, epw // LN, eb, 0)
    for f in range(F):
        pltpu.sync_copy(acc_vs[f], pacc.at[wid, f])


_NCH = 4
_NSUB = 8
_SB = 128 // _NSUB


def _heavy_body(NP, EP, src3, dst_hbm, *rest):
    tabs = rest[:_NCH]
    zrows = rest[_NCH]
    outs = rest[_NCH + 1:2 * _NCH + 1]
    sc = 2 * _NCH + 1
    acc_sp, sidx_v = rest[sc], rest[sc + 1]
    rows = rest[sc + 2:sc + 4]
    didx = rest[sc + 4:sc + 6]
    dsems = rest[sc + 6:sc + 8]
    gsems = rest[sc + 8:]
    c = lax.axis_index("c")
    s = lax.axis_index("s")
    epw = EP // NS
    nb = epw // 128
    rpt = NP // NS // 128

    pltpu.sync_copy(src3.at[s], sidx_v)

    for step in range(_NCH // NC):
        pltpu.sync_copy(zrows, rows[0])

        def zb(j, _):
            pltpu.sync_copy(rows[0], acc_sp.at[pl.ds((s * rpt + j) * 128, 128)])
            return 0

        lax.fori_loop(0, rpt, zb, 0)
        plsc.subcore_barrier()
        for cc in range(NC):
            f = NC * step + cc

            @pl.when(c == cc)
            def _():
                for b in range(2):
                    pltpu.async_copy(dst_hbm.at[pl.ds(s * epw + b * 128, 128)],
                                     didx[b], dsems[b])
                    for j in range(_NSUB):
                        pltpu.async_copy(
                            tabs[f].at[sidx_v.at[b, pl.ds(j * _SB, _SB)]],
                            rows[b].at[pl.ds(j * _SB, _SB)],
                            gsems[b * _NSUB + j])

                def outer(g, _):
                    for b in range(2):
                        i = g * 2 + b
                        pltpu.make_async_copy(
                            dst_hbm.at[pl.ds(0, 128)], didx[b], dsems[b]).wait()
                        for j in range(_NSUB):
                            pltpu.make_async_copy(
                                tabs[f].at[pl.ds(0, _SB)],
                                rows[b].at[pl.ds(j * _SB, _SB)],
                                gsems[b * _NSUB + j]).wait()
                        pltpu.sync_copy(rows[b], acc_sp.at[didx[b]], add=True)

                        @pl.when(i + 2 < nb)
                        def _():
                            nxt = i + 2
                            pltpu.async_copy(
                                dst_hbm.at[pl.ds(s * epw + nxt * 128, 128)],
                                didx[b], dsems[b])
                            for j in range(_NSUB):
                                pltpu.async_copy(
                                    tabs[f].at[sidx_v.at[nxt, pl.ds(j * _SB, _SB)]],
                                    rows[b].at[pl.ds(j * _SB, _SB)],
                                    gsems[b * _NSUB + j])
                    return 0

                lax.fori_loop(0, nb // 2, outer, 0)

        plsc.subcore_barrier()
        for cc in range(NC):
            f = NC * step + cc

            @pl.when(c == cc)
            def _():
                def ob(j, _):
                    r = (s * rpt + j) * 128
                    pltpu.sync_copy(acc_sp.at[pl.ds(r, 128)], outs[f].at[pl.ds(r, 128)])
                    return 0

                lax.fori_loop(0, rpt, ob, 0)

        plsc.subcore_barrier()


def _prep_body(degacc, cntacc, xT, dinv_row, xs, cnt):
    deg = jnp.sum(degacc[...], axis=0) + 1.0
    dr = lax.rsqrt(deg)
    dinv_row[...] = dr
    xs[...] = xT[...] * dr
    cs = jnp.sum(cntacc[...], axis=0, keepdims=True)
    cnt[...] = cs[:, :G]


def _layer1_body(pacc, xs, dinv_row, dinv_col, W1, b1r, W3, Wlr, b3r, *outs):
    houts = outs[:_NCH]
    vrow, cconst = outs[_NCH], outs[_NCH + 1]
    p = jnp.sum(pacc[...], axis=0)
    y1 = (p + xs[...]) * dinv_row[...]
    z = lax.dot_general(y1, W1[...], (((0,), (0,)), ((), ())),
                        preferred_element_type=jnp.float32)
    h1 = jnp.maximum(z + b1r[...], 0.0)
    hs = h1 * dinv_col[...]
    cw = hs.shape[1] // _NCH
    for f in range(_NCH):
        houts[f][...] = hs[:, f * cw:(f + 1) * cw]
    vrow[...] = lax.dot_general(Wlr[...], W3[...], (((1,), (1,)), ((), ())),
                                preferred_element_type=jnp.float32)
    cconst[...] = jnp.sum(b3r[...] * Wlr[...], axis=1, keepdims=True)


def _layer2_body(*args):
    ps = args[:_NCH]
    hs = args[_NCH:2 * _NCH]
    dinv_col, W2, b2r, vrow, ts = args[2 * _NCH:]
    dc = dinv_col[...]
    ys = [(ps[i][...] + hs[i][...]) * dc for i in range(_NCH)]
    y2 = jnp.concatenate(ys, axis=1)
    z = jnp.dot(y2, W2[...], preferred_element_type=jnp.float32)
    h = jnp.maximum(z + b2r[...], 0.0)
    t = jnp.sum(h * vrow[...], axis=1, keepdims=True)
    ts[...] = t * dc


def _final_body(qacc, ts_row, dinv_row, bat_col, cnt, cconst, blr, out):
    q = jnp.sum(qacc[...], axis=0)
    sv = dinv_row[...] * (q + ts_row[...]) + cconst[0, 0]
    iota = lax.broadcasted_iota(jnp.int32, (1, G), 1)
    oh = (bat_col[...] == iota).astype(jnp.float32)
    sums = jnp.dot(sv, oh, preferred_element_type=jnp.float32)
    out[...] = sums / jnp.maximum(cnt[...], 1.0) + blr[...]


def kernel(x, edge_index, batch, W1, b1, W2, b2, W3, b3, Wl, bl):
    N, IND = x.shape
    E = edge_index.shape[1]
    H = W1.shape[1]
    f32 = jnp.float32

    NP = _round_up(N + 1, 2048)
    EP = _round_up(E, 8192)
    CP = G + 32

    pad_src = (jnp.arange(EP - E, dtype=jnp.int32) * 8) % N
    srcp = jnp.concatenate([edge_index[0], pad_src])
    dstp = jnp.concatenate([edge_index[1], jnp.full((EP - E,), N, jnp.int32)])
    batp = jnp.concatenate([batch, jnp.full((NP - N,), G, jnp.int32)])
    xTp = jnp.concatenate([x.astype(f32).T,
                           jnp.zeros((IND, NP - N), f32)], axis=1)

    mesh = _sc_mesh()

    degacc, cntacc = pl.kernel(
        functools.partial(_deg_body, NP, EP, CP),
        out_type=[jax.ShapeDtypeStruct((NW, NP), f32),
                  jax.ShapeDtypeStruct((NW, CP), f32)],
        mesh=mesh,
        compiler_params=_SC_PARAMS,
        scratch_types=[
            pltpu.VMEM((EP // NW,), jnp.int32),
            pltpu.VMEM((NP // NW,), jnp.int32),
            pltpu.VMEM((NP,), f32),
            pltpu.VMEM((CP,), f32),
        ],
    )(dstp, batp)

    dinv_row, xs, cnt = pl.pallas_call(
        _prep_body,
        out_shape=[jax.ShapeDtypeStruct((1, NP), f32),
                   jax.ShapeDtypeStruct((IND, NP), f32),
                   jax.ShapeDtypeStruct((1, G), f32)],
    )(degacc.reshape(NW, 1, NP), cntacc, xTp)

    pacc = pl.kernel(
        functools.partial(_narrow_body, NP, EP, IND),
        out_type=jax.ShapeDtypeStruct((NW, IND, NP), f32),
        mesh=mesh,
        compiler_params=_SC_PARAMS,
        scratch_types=[
            pltpu.VMEM((EP // NW,), jnp.int32),
            pltpu.VMEM((EP // NW,), jnp.int32),
        ] + [pltpu.VMEM((NP,), f32) for _ in range(2 * IND)],
    )(srcp, dstp, xs)

    R = 1024
    RB = NP // R
    CW = H // _NCH
    l1_outs = pl.pallas_call(
        _layer1_body,
        grid=(RB,),
        in_specs=[
            pl.BlockSpec((NW, IND, R), lambda r: (0, 0, r)),
            pl.BlockSpec((IND, R), lambda r: (0, r)),
            pl.BlockSpec((1, R), lambda r: (0, r)),
            pl.BlockSpec((R, 1), lambda r: (r, 0)),
            pl.BlockSpec((IND, H), lambda r: (0, 0)),
            pl.BlockSpec((1, H), lambda r: (0, 0)),
            pl.BlockSpec((H, H), lambda r: (0, 0)),
            pl.BlockSpec((1, H), lambda r: (0, 0)),
            pl.BlockSpec((1, H), lambda r: (0, 0)),
        ],
        out_specs=[pl.BlockSpec((R, CW), lambda r: (r, 0)) for _ in range(_NCH)]
        + [pl.BlockSpec((1, H), lambda r: (0, 0)),
           pl.BlockSpec((1, 1), lambda r: (0, 0))],
        out_shape=[jax.ShapeDtypeStruct((NP, CW), f32) for _ in range(_NCH)]
        + [jax.ShapeDtypeStruct((1, H), f32),
           jax.ShapeDtypeStruct((1, 1), f32)],
    )(pacc, xs, dinv_row, dinv_row.reshape(NP, 1), W1.astype(f32),
      b1.reshape(1, H), W3.astype(f32), Wl.reshape(1, H), b3.reshape(1, H))
    hcs = l1_outs[:_NCH]
    vrow, cconst = l1_outs[_NCH], l1_outs[_NCH + 1]

    zrows = jnp.zeros((128, CW), f32)
    nb = EP // NS // 128
    pcs = pl.kernel(
        functools.partial(_heavy_body, NP, EP),
        out_type=[jax.ShapeDtypeStruct((NP, CW), f32) for _ in range(_NCH)],
        mesh=mesh,
        compiler_params=_SC_PARAMS,
        scratch_types=[
            pltpu.VMEM_SHARED((NP, CW), f32),
            pltpu.VMEM((nb, 128), jnp.int32),
            pltpu.VMEM((128, CW), f32),
            pltpu.VMEM((128, CW), f32),
            pltpu.VMEM((128,), jnp.int32),
            pltpu.VMEM((128,), jnp.int32),
        ] + [pltpu.SemaphoreType.DMA for _ in range(2 + 2 * _NSUB)],
    )(srcp.reshape(NS, nb, 128), dstp, *hcs, zrows)

    ts = pl.pallas_call(
        _layer2_body,
        grid=(RB,),
        in_specs=[pl.BlockSpec((R, CW), lambda r: (r, 0)) for _ in range(2 * _NCH)]
        + [
            pl.BlockSpec((R, 1), lambda r: (r, 0)),
            pl.BlockSpec((H, H), lambda r: (0, 0)),
            pl.BlockSpec((1, H), lambda r: (0, 0)),
            pl.BlockSpec((1, H), lambda r: (0, 0)),
        ],
        out_specs=pl.BlockSpec((R, 1), lambda r: (r, 0)),
        out_shape=jax.ShapeDtypeStruct((NP, 1), f32),
    )(*pcs, *hcs, dinv_row.reshape(NP, 1),
      W2.astype(f32), b2.reshape(1, H), vrow)

    qacc = pl.kernel(
        functools.partial(_narrow_body, NP, EP, 1),
        out_type=jax.ShapeDtypeStruct((NW, 1, NP), f32),
        mesh=mesh,
        compiler_params=_SC_PARAMS,
        scratch_types=[
            pltpu.VMEM((EP // NW,), jnp.int32),
            pltpu.VMEM((EP // NW,), jnp.int32),
            pltpu.VMEM((NP,), f32),
            pltpu.VMEM((NP,), f32),
        ],
    )(srcp, dstp, ts.reshape(1, NP))

    out = pl.pallas_call(
        _final_body,
        out_shape=jax.ShapeDtypeStruct((1, G), f32),
    )(qacc.reshape(NW, 1, NP), ts.reshape(1, NP), dinv_row, batp.reshape(NP, 1),
      cnt, cconst, bl.reshape(1, 1))

    return out.reshape(G, 1)


# speedup vs baseline: 1.0306x; 1.0306x over previous
import functools
import jax
import jax.numpy as jnp
from jax import lax
from jax.experimental import pallas as pl
from jax.experimental.pallas import tpu as pltpu
from jax.experimental.pallas import tpu_sc as plsc

NC = 2
NS = 16
NW = NC * NS
LN = 16
G = 128


def _round_up(a, m):
    return ((a + m - 1) // m) * m


def _sc_mesh():
    return plsc.VectorSubcoreMesh(core_axis_name="c", subcore_axis_name="s")


_SC_PARAMS = pltpu.CompilerParams(needs_layout_passes=False)


def _zero_vec(ref, n):
    z = jnp.zeros((LN,), jnp.float32)

    def body(i, _):
        ref[pl.ds(i * LN, LN)] = z
        return 0

    lax.fori_loop(0, n // LN, body, 0)


def _deg_body(NP, EP, CP, dst_hbm, bat_hbm, degacc, cntacc, dst_v, bat_v, deg_v, cnt_v):
    c = lax.axis_index("c")
    s = lax.axis_index("s")
    wid = s * NC + c
    _zero_vec(deg_v, NP)
    _zero_vec(cnt_v, CP)
    ones = jnp.ones((LN,), jnp.float32)

    epw = EP // NW
    pltpu.sync_copy(dst_hbm.at[pl.ds(wid * epw, epw)], dst_v)

    def eb(i, _):
        d16 = dst_v[pl.ds(i * LN, LN)]
        plsc.addupdate_scatter(deg_v, [d16], ones)
        return 0

    lax.fori_loop(0, epw // LN, eb, 0)

    npw = NP // NW
    pltpu.sync_copy(bat_hbm.at[pl.ds(wid * npw, npw)], bat_v)

    def bb(i, _):
        b16 = bat_v[pl.ds(i * LN, LN)]
        plsc.addupdate_scatter(cnt_v, [b16], ones)
        return 0

    lax.fori_loop(0, npw // LN, bb, 0)

    pltpu.sync_copy(deg_v, degacc.at[wid])
    pltpu.sync_copy(cnt_v, cntacc.at[wid])


def _narrow_body(NP, EP, F, src_hbm, dst_hbm, tabs_hbm, pacc, *scratch):
    src_v, dst_v = scratch[0], scratch[1]
    tab_vs = scratch[2:2 + F]
    acc_vs = scratch[2 + F:2 + 2 * F]
    c = lax.axis_index("c")
    s = lax.axis_index("s")
    wid = s * NC + c
    for f in range(F):
        pltpu.sync_copy(tabs_hbm.at[f], tab_vs[f])
        _zero_vec(acc_vs[f], NP)

    epw = EP // NW
    pltpu.sync_copy(src_hbm.at[pl.ds(wid * epw, epw)], src_v)
    pltpu.sync_copy(dst_hbm.at[pl.ds(wid * epw, epw)], dst_v)

    def eb(i, _):
        s16 = src_v[pl.ds(i * LN, LN)]
        d16 = dst_v[pl.ds(i * LN, LN)]
        for f in range(F):
            g = plsc.load_gather(tab_vs[f], [s16])
            plsc.addupdate_scatter(acc_vs[f], [d16], g)
        return 0

    lax.fori_loop(0, epw // LN, eb, 0)
    for f in range(F):
        pltpu.sync_copy(acc_vs[f], pacc.at[wid, f])


_NCH = 4
_NSUB = 4
_SB = 128 // _NSUB
_NRING = 3


def _heavy_body(NP, AR, EP, src_hbm, dst_hbm, *rest):
    tabs = rest[:_NCH]
    zrows = rest[_NCH]
    outs = rest[_NCH + 1:2 * _NCH + 1]
    sc = 2 * _NCH + 1
    acc_sp = rest[sc]
    rows = rest[sc + 1:sc + 1 + _NRING]
    sidx = rest[sc + 1 + _NRING:sc + 1 + 2 * _NRING]
    didx = rest[sc + 1 + 2 * _NRING:sc + 1 + 3 * _NRING]
    ssems = rest[sc + 1 + 3 * _NRING:sc + 1 + 4 * _NRING]
    dsems = rest[sc + 1 + 4 * _NRING:sc + 1 + 5 * _NRING]
    gsems = rest[sc + 1 + 5 * _NRING:]
    c = lax.axis_index("c")
    s = lax.axis_index("s")
    epw = EP // NS
    nb = epw // 128
    nch = AR // 128
    cpt = (nch + NS - 1) // NS

    def fire(b, i):
        base = s * epw + i * 128
        pltpu.async_copy(src_hbm.at[pl.ds(base, 128)], sidx[b], ssems[b])
        pltpu.async_copy(dst_hbm.at[pl.ds(base, 128)], didx[b], dsems[b])

    def fire_gather(tab, b):
        for j in range(_NSUB):
            pltpu.async_copy(
                tab.at[sidx[b].at[pl.ds(j * _SB, _SB)]],
                rows[b].at[pl.ds(j * _SB, _SB)],
                gsems[b * _NSUB + j])

    for step in range(_NCH // NC):
        pltpu.sync_copy(zrows, rows[0])

        def zb(j, _):
            r = s * cpt + j

            @pl.when(r < nch)
            def _():
                pltpu.sync_copy(rows[0], acc_sp.at[pl.ds(r * 128, 128)])
            return 0

        lax.fori_loop(0, cpt, zb, 0)
        plsc.subcore_barrier()
        for cc in range(NC):
            f = NC * step + cc

            @pl.when(c == cc)
            def _():
                @pl.when(s == NS - 1)
                def _():
                    pltpu.sync_copy(rows[0], outs[f].at[pl.ds(AR, NP - AR)])

                for b in range(_NRING):
                    fire(b, b)
                for b in range(_NRING):
                    pltpu.make_async_copy(
                        src_hbm.at[pl.ds(0, 128)], sidx[b], ssems[b]).wait()
                    fire_gather(tabs[f], b)

                def outer(g, _):
                    for b in range(_NRING):
                        i = g * _NRING + b

                        @pl.when(i < nb)
                        def _():
                            pltpu.make_async_copy(
                                dst_hbm.at[pl.ds(0, 128)], didx[b], dsems[b]).wait()
                            for j in range(_NSUB):
                                pltpu.make_async_copy(
                                    tabs[f].at[pl.ds(0, _SB)],
                                    rows[b].at[pl.ds(j * _SB, _SB)],
                                    gsems[b * _NSUB + j]).wait()

                            @pl.when(i + _NRING < nb)
                            def _():
                                pltpu.async_copy(
                                    src_hbm.at[pl.ds(s * epw + (i + _NRING) * 128, 128)],
                                    sidx[b], ssems[b])

                            pltpu.sync_copy(rows[b], acc_sp.at[didx[b]], add=True)

                        @pl.when(i + _NRING < nb)
                        def _():
                            pltpu.async_copy(
                                dst_hbm.at[pl.ds(s * epw + (i + _NRING) * 128, 128)],
                                didx[b], dsems[b])
                            pltpu.make_async_copy(
                                src_hbm.at[pl.ds(0, 128)], sidx[b], ssems[b]).wait()
                            fire_gather(tabs[f], b)
                    return 0

                lax.fori_loop(0, (nb + _NRING - 1) // _NRING, outer, 0)

        plsc.subcore_barrier()
        for cc in range(NC):
            f = NC * step + cc

            @pl.when(c == cc)
            def _():
                def ob(j, _):
                    r = s * cpt + j

                    @pl.when(r < nch)
                    def _():
                        pltpu.sync_copy(acc_sp.at[pl.ds(r * 128, 128)],
                                        outs[f].at[pl.ds(r * 128, 128)])
                    return 0

                lax.fori_loop(0, cpt, ob, 0)

        plsc.subcore_barrier()


def _prep_body(degacc, cntacc, xT, dinv_row, xs, cnt):
    deg = jnp.sum(degacc[...], axis=0) + 1.0
    dr = lax.rsqrt(deg)
    dinv_row[...] = dr
    xs[...] = xT[...] * dr
    cs = jnp.sum(cntacc[...], axis=0, keepdims=True)
    cnt[...] = cs[:, :G]


def _layer1_body(pacc, xs, dinv_row, dinv_col, W1, b1r, W3, Wlr, b3r, *outs):
    houts = outs[:_NCH]
    vrow, cconst = outs[_NCH], outs[_NCH + 1]
    p = jnp.sum(pacc[...], axis=0)
    y1 = (p + xs[...]) * dinv_row[...]
    z = lax.dot_general(y1, W1[...], (((0,), (0,)), ((), ())),
                        preferred_element_type=jnp.float32)
    h1 = jnp.maximum(z + b1r[...], 0.0)
    hs = h1 * dinv_col[...]
    cw = hs.shape[1] // _NCH
    for f in range(_NCH):
        houts[f][...] = hs[:, f * cw:(f + 1) * cw]
    vrow[...] = lax.dot_general(Wlr[...], W3[...], (((1,), (1,)), ((), ())),
                                preferred_element_type=jnp.float32)
    cconst[...] = jnp.sum(b3r[...] * Wlr[...], axis=1, keepdims=True)


def _layer2_body(*args):
    ps = args[:_NCH]
    hs = args[_NCH:2 * _NCH]
    dinv_col, W2, b2r, vrow, ts = args[2 * _NCH:]
    dc = dinv_col[...]
    ys = [(ps[i][...] + hs[i][...]) * dc for i in range(_NCH)]
    y2 = jnp.concatenate(ys, axis=1)
    z = jnp.dot(y2, W2[...], preferred_element_type=jnp.float32)
    h = jnp.maximum(z + b2r[...], 0.0)
    t = jnp.sum(h * vrow[...], axis=1, keepdims=True)
    ts[...] = t * dc


def _final_body(qacc, ts_row, dinv_row, bat_col, cnt, cconst, blr, out):
    q = jnp.sum(qacc[...], axis=0)
    sv = dinv_row[...] * (q + ts_row[...]) + cconst[0, 0]
    iota = lax.broadcasted_iota(jnp.int32, (1, G), 1)
    oh = (bat_col[...] == iota).astype(jnp.float32)
    sums = jnp.dot(sv, oh, preferred_element_type=jnp.float32)
    out[...] = sums / jnp.maximum(cnt[...], 1.0) + blr[...]


def kernel(x, edge_index, batch, W1, b1, W2, b2, W3, b3, Wl, bl):
    N, IND = x.shape
    E = edge_index.shape[1]
    H = W1.shape[1]
    f32 = jnp.float32

    NP = _round_up(N + 1, 2048)
    EP = _round_up(E, 8192)
    CP = G + 32

    pad_src = (jnp.arange(EP - E, dtype=jnp.int32) * 8) % N
    srcp = jnp.concatenate([edge_index[0], pad_src])
    dstp = jnp.concatenate([edge_index[1], jnp.full((EP - E,), N, jnp.int32)])
    batp = jnp.concatenate([batch, jnp.full((NP - N,), G, jnp.int32)])
    xTp = jnp.concatenate([x.astype(f32).T,
                           jnp.zeros((IND, NP - N), f32)], axis=1)

    mesh = _sc_mesh()

    degacc, cntacc = pl.kernel(
        functools.partial(_deg_body, NP, EP, CP),
        out_type=[jax.ShapeDtypeStruct((NW, NP), f32),
                  jax.ShapeDtypeStruct((NW, CP), f32)],
        mesh=mesh,
        compiler_params=_SC_PARAMS,
        scratch_types=[
            pltpu.VMEM((EP // NW,), jnp.int32),
            pltpu.VMEM((NP // NW,), jnp.int32),
            pltpu.VMEM((NP,), f32),
            pltpu.VMEM((CP,), f32),
        ],
    )(dstp, batp)

    dinv_row, xs, cnt = pl.pallas_call(
        _prep_body,
        out_shape=[jax.ShapeDtypeStruct((1, NP), f32),
                   jax.ShapeDtypeStruct((IND, NP), f32),
                   jax.ShapeDtypeStruct((1, G), f32)],
    )(degacc.reshape(NW, 1, NP), cntacc, xTp)

    pacc = pl.kernel(
        functools.partial(_narrow_body, NP, EP, IND),
        out_type=jax.ShapeDtypeStruct((NW, IND, NP), f32),
        mesh=mesh,
        compiler_params=_SC_PARAMS,
        scratch_types=[
            pltpu.VMEM((EP // NW,), jnp.int32),
            pltpu.VMEM((EP // NW,), jnp.int32),
        ] + [pltpu.VMEM((NP,), f32) for _ in range(2 * IND)],
    )(srcp, dstp, xs)

    R = 1024
    RB = NP // R
    CW = H // _NCH
    l1_outs = pl.pallas_call(
        _layer1_body,
        grid=(RB,),
        in_specs=[
            pl.BlockSpec((NW, IND, R), lambda r: (0, 0, r)),
            pl.BlockSpec((IND, R), lambda r: (0, r)),
            pl.BlockSpec((1, R), lambda r: (0, r)),
            pl.BlockSpec((R, 1), lambda r: (r, 0)),
            pl.BlockSpec((IND, H), lambda r: (0, 0)),
            pl.BlockSpec((1, H), lambda r: (0, 0)),
            pl.BlockSpec((H, H), lambda r: (0, 0)),
            pl.BlockSpec((1, H), lambda r: (0, 0)),
            pl.BlockSpec((1, H), lambda r: (0, 0)),
        ],
        out_specs=[pl.BlockSpec((R, CW), lambda r: (r, 0)) for _ in range(_NCH)]
        + [pl.BlockSpec((1, H), lambda r: (0, 0)),
           pl.BlockSpec((1, 1), lambda r: (0, 0))],
        out_shape=[jax.ShapeDtypeStruct((NP, CW), f32) for _ in range(_NCH)]
        + [jax.ShapeDtypeStruct((1, H), f32),
           jax.ShapeDtypeStruct((1, 1), f32)],
    )(pacc, xs, dinv_row, dinv_row.reshape(NP, 1), W1.astype(f32),
      b1.reshape(1, H), W3.astype(f32), Wl.reshape(1, H), b3.reshape(1, H))
    hcs = l1_outs[:_NCH]
    vrow, cconst = l1_outs[_NCH], l1_outs[_NCH + 1]

    zrows = jnp.zeros((128, CW), f32)
    AR = _round_up(N + 1, 128)
    pcs = pl.kernel(
        functools.partial(_heavy_body, NP, AR, EP),
        out_type=[jax.ShapeDtypeStruct((NP, CW), f32) for _ in range(_NCH)],
        mesh=mesh,
        compiler_params=_SC_PARAMS,
        scratch_types=[
            pltpu.VMEM_SHARED((AR, CW), f32),
        ] + [pltpu.VMEM((128, CW), f32) for _ in range(_NRING)]
        + [pltpu.VMEM((128,), jnp.int32) for _ in range(2 * _NRING)]
        + [pltpu.SemaphoreType.DMA for _ in range(2 * _NRING + _NRING * _NSUB)],
    )(srcp, dstp, *hcs, zrows)

    ts = pl.pallas_call(
        _layer2_body,
        grid=(RB,),
        in_specs=[pl.BlockSpec((R, CW), lambda r: (r, 0)) for _ in range(2 * _NCH)]
        + [
            pl.BlockSpec((R, 1), lambda r: (r, 0)),
            pl.BlockSpec((H, H), lambda r: (0, 0)),
            pl.BlockSpec((1, H), lambda r: (0, 0)),
            pl.BlockSpec((1, H), lambda r: (0, 0)),
        ],
        out_specs=pl.BlockSpec((R, 1), lambda r: (r, 0)),
        out_shape=jax.ShapeDtypeStruct((NP, 1), f32),
    )(*pcs, *hcs, dinv_row.reshape(NP, 1),
      W2.astype(f32), b2.reshape(1, H), vrow)

    qacc = pl.kernel(
        functools.partial(_narrow_body, NP, EP, 1),
        out_type=jax.ShapeDtypeStruct((NW, 1, NP), f32),
        mesh=mesh,
        compiler_params=_SC_PARAMS,
        scratch_types=[
            pltpu.VMEM((EP // NW,), jnp.int32),
            pltpu.VMEM((EP // NW,), jnp.int32),
            pltpu.VMEM((NP,), f32),
            pltpu.VMEM((NP,), f32),
        ],
    )(srcp, dstp, ts.reshape(1, NP))

    out = pl.pallas_call(
        _final_body,
        out_shape=jax.ShapeDtypeStruct((1, G), f32),
    )(qacc.reshape(NW, 1, NP), ts.reshape(1, NP), dinv_row, batp.reshape(NP, 1),
      cnt, cconst, bl.reshape(1, 1))

    return out.reshape(G, 1)
